# Initial kernel scaffold; baseline (speedup 1.0000x reference)
import jax
import jax.numpy as jnp
from jax.experimental import pallas as pl


def kernel(hidden_states, weight):
    raise NotImplementedError("write your pallas kernel here")


# speedup vs baseline: 14.7040x; 14.7040x over previous
import functools

import jax
import jax.numpy as jnp
from jax.experimental import pallas as pl
from jax.experimental.pallas import tpu as pltpu

_BSZ, _SEQ, _H, _E = 4, 4096, 256, 32
_TOP_K = 1
_ALPHA = 0.001
_SCALE = 1.0
_R = 2048
_NBLK = (_BSZ * _SEQ) // _R
_BPB = _SEQ // _R


def _gate_body(x_ref, w_ref, idx_ref, wgt_ref, aux_ref, cnt_ref, ssum_ref):
    i = pl.program_id(0)

    @pl.when(i == 0)
    def _init():
        cnt_ref[...] = jnp.zeros_like(cnt_ref)
        ssum_ref[...] = jnp.zeros_like(ssum_ref)

    x = x_ref[...]
    w = w_ref[...]
    logits = jax.lax.dot_general(
        w, x, (((1,), (1,)), ((), ())), preferred_element_type=jnp.float32
    )
    colmax = jnp.max(logits, axis=0, keepdims=True)
    ex = jnp.exp(logits - colmax)
    denom = jnp.sum(ex, axis=0, keepdims=True)
    eiota = jax.lax.broadcasted_iota(jnp.int32, logits.shape, 0)
    idx = jnp.min(jnp.where(logits == colmax, eiota, _E), axis=0, keepdims=True)
    idx_ref[...] = idx.reshape(1, 1, _R)
    wgt_ref[...] = (_SCALE / denom).reshape(1, 1, _R)

    b = i // _BPB
    probs_sum = jnp.sum(ex / denom, axis=1, keepdims=True)
    onehot = (eiota == idx).astype(jnp.float32)
    cnt = jnp.sum(onehot, axis=1, keepdims=True)
    liota = jax.lax.broadcasted_iota(jnp.int32, (1, 128), 1)
    bmask = (liota == b).astype(jnp.float32)
    cnt_ref[...] += cnt * bmask
    ssum_ref[...] += probs_sum * bmask

    @pl.when(i == _NBLK - 1)
    def _fin():
        tot = jnp.sum(cnt_ref[:, : _BSZ] * ssum_ref[:, : _BSZ], keepdims=True)
        aux_ref[...] = tot.reshape(1, 1) * (_ALPHA * _E / (_SEQ * float(_SEQ) * _BSZ))


@functools.partial(jax.jit, static_argnames=())
def kernel(hidden_states, weight):
    x = hidden_states.reshape(-1, _H)
    idxs, wgts, aux = pl.pallas_call(
        _gate_body,
        grid=(_NBLK,),
        in_specs=[
            pl.BlockSpec((_R, _H), lambda i: (i, 0)),
            pl.BlockSpec((_E, _H), lambda i: (0, 0)),
        ],
        out_specs=[
            pl.BlockSpec((1, 1, _R), lambda i: (i, 0, 0)),
            pl.BlockSpec((1, 1, _R), lambda i: (i, 0, 0)),
            pl.BlockSpec((1, 1), lambda i: (0, 0)),
        ],
        out_shape=[
            jax.ShapeDtypeStruct((_NBLK, 1, _R), jnp.int32),
            jax.ShapeDtypeStruct((_NBLK, 1, _R), jnp.float32),
            jax.ShapeDtypeStruct((1, 1), jnp.float32),
        ],
        scratch_shapes=[
            pltpu.VMEM((_E, 128), jnp.float32),
            pltpu.VMEM((_E, 128), jnp.float32),
        ],
    )(x, weight)
    topk_idx = idxs.reshape(-1, _TOP_K)
    topk_weight = wgts.reshape(-1, _TOP_K)
    return (topk_idx, topk_weight, aux[0, 0])


# speedup vs baseline: 18.0641x; 1.2285x over previous
import functools

import jax
import jax.numpy as jnp
from jax.experimental import pallas as pl
from jax.experimental.pallas import tpu as pltpu

_BSZ, _SEQ, _H, _E = 4, 4096, 256, 32
_TOP_K = 1
_ALPHA = 0.001
_SCALE = 1.0
_R = 4096
_NBLK = (_BSZ * _SEQ) // _R
_BPB = _SEQ // _R


def _gate_body(x_ref, w_ref, idx_ref, wgt_ref, aux_ref, cnt_ref, ssum_ref):
    i = pl.program_id(0)

    @pl.when(i == 0)
    def _init():
        cnt_ref[...] = jnp.zeros_like(cnt_ref)
        ssum_ref[...] = jnp.zeros_like(ssum_ref)

    x = x_ref[...]
    w = w_ref[...]
    logits = jax.lax.dot_general(
        w, x, (((1,), (1,)), ((), ())), preferred_element_type=jnp.float32
    )
    colmax = jnp.max(logits, axis=0, keepdims=True)
    ex = jnp.exp(logits - colmax)
    denom = jnp.sum(ex, axis=0, keepdims=True)
    eiota = jax.lax.broadcasted_iota(jnp.int32, logits.shape, 0)
    idx = jnp.min(jnp.where(logits == colmax, eiota, _E), axis=0, keepdims=True)
    idx_ref[...] = idx.reshape(1, 1, _R)
    wgt_ref[...] = (_SCALE / denom).reshape(1, 1, _R)

    b = i // _BPB
    probs_sum = jnp.sum(ex / denom, axis=1, keepdims=True)
    onehot = (eiota == idx).astype(jnp.float32)
    cnt = jnp.sum(onehot, axis=1, keepdims=True)
    liota = jax.lax.broadcasted_iota(jnp.int32, (1, 128), 1)
    bmask = (liota == b).astype(jnp.float32)
    cnt_ref[...] += cnt * bmask
    ssum_ref[...] += probs_sum * bmask

    @pl.when(i == _NBLK - 1)
    def _fin():
        tot = jnp.sum(cnt_ref[:, : _BSZ] * ssum_ref[:, : _BSZ], keepdims=True)
        aux_ref[...] = tot.reshape(1, 1) * (_ALPHA * _E / (_SEQ * float(_SEQ) * _BSZ))


@functools.partial(jax.jit, static_argnames=())
def kernel(hidden_states, weight):
    x = hidden_states.reshape(-1, _H)
    idxs, wgts, aux = pl.pallas_call(
        _gate_body,
        grid=(_NBLK,),
        in_specs=[
            pl.BlockSpec((_R, _H), lambda i: (i, 0)),
            pl.BlockSpec((_E, _H), lambda i: (0, 0)),
        ],
        out_specs=[
            pl.BlockSpec((1, 1, _R), lambda i: (i, 0, 0)),
            pl.BlockSpec((1, 1, _R), lambda i: (i, 0, 0)),
            pl.BlockSpec((1, 1), lambda i: (0, 0)),
        ],
        out_shape=[
            jax.ShapeDtypeStruct((_NBLK, 1, _R), jnp.int32),
            jax.ShapeDtypeStruct((_NBLK, 1, _R), jnp.float32),
            jax.ShapeDtypeStruct((1, 1), jnp.float32),
        ],
        scratch_shapes=[
            pltpu.VMEM((_E, 128), jnp.float32),
            pltpu.VMEM((_E, 128), jnp.float32),
        ],
    )(x, weight)
    topk_idx = idxs.reshape(-1, _TOP_K)
    topk_weight = wgts.reshape(-1, _TOP_K)
    return (topk_idx, topk_weight, aux[0, 0])


# speedup vs baseline: 18.6934x; 1.0348x over previous
import functools

import jax
import jax.numpy as jnp
from jax.experimental import pallas as pl
from jax.experimental.pallas import tpu as pltpu

_BSZ, _SEQ, _H, _E = 4, 4096, 256, 32
_TOP_K = 1
_ALPHA = 0.001
_SCALE = 1.0
_R = 8192
_NBLK = (_BSZ * _SEQ) // _R


def _gate_body(x_ref, w_ref, idx_ref, wgt_ref, aux_ref, cnt_ref, ssum_ref):
    i = pl.program_id(0)

    @pl.when(i == 0)
    def _init():
        cnt_ref[...] = jnp.zeros_like(cnt_ref)
        ssum_ref[...] = jnp.zeros_like(ssum_ref)

    x = x_ref[...]
    w = w_ref[...]
    logits = jax.lax.dot_general(
        w, x, (((1,), (1,)), ((), ())), preferred_element_type=jnp.float32
    )
    colmax = jnp.max(logits, axis=0, keepdims=True)
    ex = jnp.exp(logits - colmax)
    denom = jnp.sum(ex, axis=0, keepdims=True)
    eiota = jax.lax.broadcasted_iota(jnp.int32, logits.shape, 0)
    idx = jnp.min(jnp.where(logits == colmax, eiota, _E), axis=0, keepdims=True)
    idx_ref[...] = idx.reshape(1, 1, _R)
    wgt_ref[...] = (_SCALE / denom).reshape(1, 1, _R)

    probs = ex / denom
    onehot = (eiota == idx).astype(jnp.float32)
    liota = jax.lax.broadcasted_iota(jnp.int32, (1, 128), 1)
    if _R <= _SEQ:
        b = i // (_SEQ // _R)
        bmask = (liota == b).astype(jnp.float32)
        cnt_ref[...] += jnp.sum(onehot, axis=1, keepdims=True) * bmask
        ssum_ref[...] += jnp.sum(probs, axis=1, keepdims=True) * bmask
    else:
        for j in range(_R // _SEQ):
            b = i * (_R // _SEQ) + j
            seg = slice(j * _SEQ, (j + 1) * _SEQ)
            bmask = (liota == b).astype(jnp.float32)
            cnt_ref[...] += jnp.sum(onehot[:, seg], axis=1, keepdims=True) * bmask
            ssum_ref[...] += jnp.sum(probs[:, seg], axis=1, keepdims=True) * bmask

    @pl.when(i == _NBLK - 1)
    def _fin():
        tot = jnp.sum(cnt_ref[:, : _BSZ] * ssum_ref[:, : _BSZ], keepdims=True)
        aux_ref[...] = tot.reshape(1, 1) * (_ALPHA * _E / (_SEQ * float(_SEQ) * _BSZ))


@functools.partial(jax.jit, static_argnames=())
def kernel(hidden_states, weight):
    x = hidden_states.reshape(-1, _H)
    idxs, wgts, aux = pl.pallas_call(
        _gate_body,
        grid=(_NBLK,),
        in_specs=[
            pl.BlockSpec((_R, _H), lambda i: (i, 0)),
            pl.BlockSpec((_E, _H), lambda i: (0, 0)),
        ],
        out_specs=[
            pl.BlockSpec((1, 1, _R), lambda i: (i, 0, 0)),
            pl.BlockSpec((1, 1, _R), lambda i: (i, 0, 0)),
            pl.BlockSpec((1, 1), lambda i: (0, 0)),
        ],
        out_shape=[
            jax.ShapeDtypeStruct((_NBLK, 1, _R), jnp.int32),
            jax.ShapeDtypeStruct((_NBLK, 1, _R), jnp.float32),
            jax.ShapeDtypeStruct((1, 1), jnp.float32),
        ],
        scratch_shapes=[
            pltpu.VMEM((_E, 128), jnp.float32),
            pltpu.VMEM((_E, 128), jnp.float32),
        ],
    )(x, weight)
    topk_idx = idxs.reshape(-1, _TOP_K)
    topk_weight = wgts.reshape(-1, _TOP_K)
    return (topk_idx, topk_weight, aux[0, 0])


# speedup vs baseline: 18.7541x; 1.0033x over previous
import jax
import jax.numpy as jnp
from jax.experimental import pallas as pl
from jax.experimental.pallas import tpu as pltpu

_BSZ, _SEQ, _H, _E = 4, 4096, 256, 32
_TOP_K = 1
_ALPHA = 0.001
_SCALE = 1.0
_NROW = _BSZ * _SEQ

_G = 2
_S = 2
_C = _NROW // (_G * _S)
_RSTEP = _C * _S


def _gate_body(*refs):
    x_refs = refs[:_S]
    w_ref = refs[_S]
    idx_ref, wgt_ref, aux_ref, cnt_ref, ssum_ref = refs[_S + 1:]
    i = pl.program_id(0)

    @pl.when(i == 0)
    def _init():
        cnt_ref[...] = jnp.zeros_like(cnt_ref)
        ssum_ref[...] = jnp.zeros_like(ssum_ref)

    w = w_ref[...]
    liota = jax.lax.broadcasted_iota(jnp.int32, (1, 128), 1)
    for s in range(_S):
        x = x_refs[s][...]
        logits = jax.lax.dot_general(
            w, x, (((1,), (1,)), ((), ())), preferred_element_type=jnp.float32
        )
        colmax = jnp.max(logits, axis=0, keepdims=True)
        ex = jnp.exp(logits - colmax)
        denom = jnp.sum(ex, axis=0, keepdims=True)
        eiota = jax.lax.broadcasted_iota(jnp.int32, logits.shape, 0)
        idx = jnp.min(jnp.where(logits == colmax, eiota, _E), axis=0, keepdims=True)
        sl = slice(s * _C, (s + 1) * _C)
        idx_ref[0, 0, sl] = idx[0]
        wgt_ref[0, 0, sl] = (_SCALE / denom)[0]

        probs = ex / denom
        onehot = (eiota == idx).astype(jnp.float32)
        r = i * _S + s
        if _C <= _SEQ:
            b = r // (_SEQ // _C)
            bmask = (liota == b).astype(jnp.float32)
            cnt_ref[...] += jnp.sum(onehot, axis=1, keepdims=True) * bmask
            ssum_ref[...] += jnp.sum(probs, axis=1, keepdims=True) * bmask
        else:
            for j in range(_C // _SEQ):
                b = r * (_C // _SEQ) + j
                seg = slice(j * _SEQ, (j + 1) * _SEQ)
                bmask = (liota == b).astype(jnp.float32)
                cnt_ref[...] += jnp.sum(onehot[:, seg], axis=1, keepdims=True) * bmask
                ssum_ref[...] += jnp.sum(probs[:, seg], axis=1, keepdims=True) * bmask

    @pl.when(i == _G - 1)
    def _fin():
        tot = jnp.sum(cnt_ref[:, : _BSZ] * ssum_ref[:, : _BSZ], keepdims=True)
        aux_ref[...] = tot.reshape(1, 1) * (_ALPHA * _E / (_SEQ * float(_SEQ) * _BSZ))


def _mk_x_spec(s):
    return pl.BlockSpec((_C, _H), lambda i, s=s: (i * _S + s, 0))


def kernel(hidden_states, weight):
    x = hidden_states.reshape(-1, _H)
    idxs, wgts, aux = pl.pallas_call(
        _gate_body,
        grid=(_G,),
        in_specs=[_mk_x_spec(s) for s in range(_S)]
        + [pl.BlockSpec((_E, _H), lambda i: (0, 0))],
        out_specs=[
            pl.BlockSpec((1, 1, _RSTEP), lambda i: (i, 0, 0)),
            pl.BlockSpec((1, 1, _RSTEP), lambda i: (i, 0, 0)),
            pl.BlockSpec((1, 1), lambda i: (0, 0)),
        ],
        out_shape=[
            jax.ShapeDtypeStruct((_G, 1, _RSTEP), jnp.int32),
            jax.ShapeDtypeStruct((_G, 1, _RSTEP), jnp.float32),
            jax.ShapeDtypeStruct((1, 1), jnp.float32),
        ],
        scratch_shapes=[
            pltpu.VMEM((_E, 128), jnp.float32),
            pltpu.VMEM((_E, 128), jnp.float32),
        ],
    )(*([x] * _S), weight)
    topk_idx = idxs.reshape(-1, _TOP_K)
    topk_weight = wgts.reshape(-1, _TOP_K)
    return (topk_idx, topk_weight, aux[0, 0])
